# baseline (speedup 1.0000x reference)
import functools

import jax
import jax.numpy as jnp
from jax import lax
from jax.experimental import pallas as pl
from jax.experimental.pallas import tpu as pltpu
from jax.experimental.pallas import tpu_sc as plsc

_H = 64
_SEQ = 50
_NW = 32
_CHUNK = 128
_POS_REP = 200
_L = 16
_NBUF = 5
_LOOK = 3
_NSPLIT = 4


def _emb_body(ids_ref, tab_ref, pos_ref, out_ref, idx_v, pos_raw, pos_v,
              rows_v, gsem, osem):
    nchunk = ids_ref.shape[1]
    wid = lax.axis_index("s") * 2 + lax.axis_index("c")
    base = wid * (nchunk * _CHUNK)

    pltpu.sync_copy(ids_ref.at[wid], idx_v)
    pltpu.sync_copy(pos_ref, pos_raw)

    def rep_body(r, carry):
        for j in range(_H // _L):
            sl = pl.ds(j * _L, _L)
            v = pos_raw[r, sl]
            for k in range(_POS_REP // _SEQ):
                pos_v[k * _SEQ + r, sl] = v
        return carry

    lax.fori_loop(0, _SEQ, rep_body, 0)

    def start_gather(c, b):
        sub = _CHUNK // _NSPLIT
        for t in range(_NSPLIT):
            pltpu.async_copy(tab_ref.at[idx_v.at[c, pl.ds(t * sub, sub)]],
                             rows_v.at[b, pl.ds(t * sub, sub)], gsem.at[b])

    def wait_gather(c, b):
        pltpu.make_async_copy(tab_ref.at[idx_v.at[c]], rows_v.at[b],
                              gsem.at[b]).wait()

    def wait_write(b):
        pltpu.make_async_copy(rows_v.at[b], out_ref.at[pl.ds(0, _CHUNK)],
                              osem.at[b]).wait()

    for b in range(_LOOK):
        start_gather(b, b)

    def outer(g, carry):
        for b in range(_NBUF):
            c = g * _NBUF + b
            wait_gather(c, b)
            phi = lax.rem(c * _CHUNK, _SEQ)

            def add_body(r, carry2):
                p = phi + r
                for j in range(_H // _L):
                    sl = pl.ds(j * _L, _L)
                    rows_v[b, r, sl] = rows_v[b, r, sl] + pos_v[p, sl]
                return carry2

            lax.fori_loop(0, _CHUNK, add_body, 0)

            pltpu.async_copy(rows_v.at[b],
                             out_ref.at[pl.ds(base + c * _CHUNK, _CHUNK)],
                             osem.at[b])
            b3 = (b + _LOOK) % _NBUF

            @pl.when(c + _LOOK < nchunk)
            def _():
                @pl.when(c >= _NBUF - _LOOK)
                def _():
                    wait_write(b3)
                start_gather(c + _LOOK, b3)
        return carry

    lax.fori_loop(0, nchunk // _NBUF, outer, 0)
    for b in range(_NBUF):
        wait_write(b)


def kernel(input_ids, token_table, pos_table):
    b, s = input_ids.shape
    h = token_table.shape[1]
    total = b * s
    nchunk = total // (_NW * _CHUNK)
    ids = input_ids.astype(jnp.int32).reshape(_NW, nchunk, _CHUNK)
    mesh = plsc.VectorSubcoreMesh(core_axis_name="c", subcore_axis_name="s")
    run = functools.partial(
        pl.kernel,
        mesh=mesh,
        compiler_params=pltpu.CompilerParams(use_tc_tiling_on_sc=False),
        out_type=jax.ShapeDtypeStruct((total, h), jnp.float32),
        scratch_types=[
            pltpu.VMEM((nchunk, _CHUNK), jnp.int32),
            pltpu.VMEM((64, h), jnp.float32),
            pltpu.VMEM((_POS_REP, h), jnp.float32),
            pltpu.VMEM((_NBUF, _CHUNK, h), jnp.float32),
            pltpu.SemaphoreType.DMA((_NBUF,)),
            pltpu.SemaphoreType.DMA((_NBUF,)),
        ],
    )(_emb_body)
    out = run(ids, token_table, pos_table)
    return out.reshape(b, s, h)


# speedup vs baseline: 1.1280x; 1.1280x over previous
import functools

import jax
import jax.numpy as jnp
from jax import lax
from jax.experimental import pallas as pl
from jax.experimental.pallas import tpu as pltpu
from jax.experimental.pallas import tpu_sc as plsc

_H = 64
_SEQ = 50
_NW = 32
_L = 16
_NBUF = 8
_LOOK = 6


def _emb_body(ids_ref, tab_ref, pos_ref, out_ref, idx_v, pos_v, rows_v,
              gsem, osem):
    nchunk = ids_ref.shape[1]
    wid = lax.axis_index("s") * 2 + lax.axis_index("c")
    base = wid * nchunk

    pltpu.sync_copy(ids_ref.at[wid], idx_v)
    pltpu.sync_copy(pos_ref, pos_v)

    def start_gather(c, b):
        pltpu.async_copy(tab_ref.at[idx_v.at[c]], rows_v.at[b], gsem.at[b])

    def wait_gather(c, b):
        pltpu.make_async_copy(tab_ref.at[idx_v.at[c]], rows_v.at[b],
                              gsem.at[b]).wait()

    def wait_write(b):
        pltpu.make_async_copy(rows_v.at[b], out_ref.at[0], osem.at[b]).wait()

    for b in range(_LOOK):
        start_gather(b, b)

    def outer(g, carry):
        for b in range(_NBUF):
            c = g * _NBUF + b
            wait_gather(c, b)

            def add_body(r, carry2):
                for j in range(_H // _L):
                    sl = pl.ds(j * _L, _L)
                    rows_v[b, r, sl] = rows_v[b, r, sl] + pos_v[r, sl]
                return carry2

            lax.fori_loop(0, _SEQ, add_body, 0)
            pltpu.async_copy(rows_v.at[b], out_ref.at[base + c], osem.at[b])
            b3 = (b + _LOOK) % _NBUF

            @pl.when(c + _LOOK < nchunk)
            def _():
                @pl.when(c >= _NBUF - _LOOK)
                def _():
                    wait_write(b3)
                start_gather(c + _LOOK, b3)
        return carry

    lax.fori_loop(0, nchunk // _NBUF, outer, 0)
    for b in range(_NBUF):
        wait_write(b)


def kernel(input_ids, token_table, pos_table):
    b, s = input_ids.shape
    h = token_table.shape[1]
    nchunk = b // _NW
    ids = input_ids.astype(jnp.int32).reshape(_NW, nchunk, s)
    mesh = plsc.VectorSubcoreMesh(core_axis_name="c", subcore_axis_name="s")
    run = functools.partial(
        pl.kernel,
        mesh=mesh,
        compiler_params=pltpu.CompilerParams(use_tc_tiling_on_sc=False),
        out_type=jax.ShapeDtypeStruct((b, s, h), jnp.float32),
        scratch_types=[
            pltpu.VMEM((nchunk, s), jnp.int32),
            pltpu.VMEM((64, h), jnp.float32),
            pltpu.VMEM((_NBUF, s, h), jnp.float32),
            pltpu.SemaphoreType.DMA((_NBUF,)),
            pltpu.SemaphoreType.DMA((_NBUF,)),
        ],
    )(_emb_body)
    return run(ids, token_table, pos_table)
